# baseline (speedup 1.0000x reference)
import functools

import jax
import jax.numpy as jnp
from jax import lax
from jax.experimental import pallas as pl
from jax.experimental.pallas import tpu as pltpu
from jax.experimental.pallas import tpu_sc as plsc

_B, _N, _D = 16, 512, 256
_NC = 2
_NS = 16
_LANES = 16
_CHUNK = 32768
_GROUPS = _CHUNK // _LANES
_CHUNKS_PER_TILE = (_N * _N) // _CHUNK
_ROWS_PER_CHUNK = _CHUNK // _N
_GROUPS_PER_ROW = _N // _LANES


def _edge_index_body(out_hbm, buf_a, buf_b, sem_a, sem_b):
    wid = lax.axis_index("s") * _NC + lax.axis_index("c")
    r = wid // _B
    b = wid % _B
    is1 = r
    base_off = wid * (_N * _N)
    lane = lax.iota(jnp.int32, _LANES)
    bn = b * _N

    def fill(buf, c):
        def body(g, carry):
            base0 = bn + c * _ROWS_PER_CHUNK + g // _GROUPS_PER_ROW
            base1 = bn + (g % _GROUPS_PER_ROW) * _LANES
            base = jnp.where(is1 == 1, base1, base0)
            buf[pl.ds(g * _LANES, _LANES)] = (
                jnp.full((_LANES,), base, jnp.int32) + lane * is1
            )
            return carry
        lax.fori_loop(0, _GROUPS, body, 0)

    copies = []
    for c in range(_CHUNKS_PER_TILE):
        buf, sem = (buf_a, sem_a) if c % 2 == 0 else (buf_b, sem_b)
        if c >= 2:
            copies[c - 2].wait()
        fill(buf, c)
        cp = pltpu.make_async_copy(
            buf, out_hbm.at[pl.ds(base_off + c * _CHUNK, _CHUNK)], sem)
        cp.start()
        copies.append(cp)
    copies[-2].wait()
    copies[-1].wait()


@functools.lru_cache(maxsize=1)
def _edge_index_sc():
    return functools.partial(
        pl.kernel,
        out_type=jax.ShapeDtypeStruct((2 * _B * _N * _N,), jnp.int32),
        mesh=plsc.VectorSubcoreMesh(core_axis_name="c", subcore_axis_name="s"),
        scratch_types=[
            pltpu.VMEM((_CHUNK,), jnp.int32),
            pltpu.VMEM((_CHUNK,), jnp.int32),
            pltpu.SemaphoreType.DMA,
            pltpu.SemaphoreType.DMA,
        ],
    )(_edge_index_body)


def _edge_attr_body(adj_ref, adjt_ref, out_ref, sym_ref):
    @pl.when(pl.program_id(0) == 0)
    def _():
        a = 1.0 / (1.0 + jnp.exp(-adj_ref[...]))
        at = 1.0 / (1.0 + jnp.exp(-adjt_ref[...]))
        sym_ref[...] = (a + at) * 0.5

    out_ref[...] = sym_ref[...]


_edge_attr_tc = pl.pallas_call(
    _edge_attr_body,
    grid=(_B,),
    in_specs=[
        pl.BlockSpec((_N, _N), lambda b: (0, 0)),
        pl.BlockSpec((_N, _N), lambda b: (0, 0)),
    ],
    out_specs=pl.BlockSpec((_N, _N), lambda b: (b, 0)),
    out_shape=jax.ShapeDtypeStruct((_B * _N, _N), jnp.float32),
    scratch_shapes=[pltpu.VMEM((_N, _N), jnp.float32)],
)


def kernel(x, adj):
    Bv, Nv, Dv = x.shape
    edge_index = _edge_index_sc()().reshape(2, Bv * Nv * Nv)
    edge_attr = _edge_attr_tc(adj, adj.T).reshape(-1)
    x_batched = x.reshape(Bv * Nv, Dv)
    return x_batched, edge_index, edge_attr


# speedup vs baseline: 11.0105x; 11.0105x over previous
import functools

import jax
import jax.numpy as jnp
from jax import lax
from jax.experimental import pallas as pl
from jax.experimental.pallas import tpu as pltpu
from jax.experimental.pallas import tpu_sc as plsc

_B, _N, _D = 16, 512, 256
_NC = 2
_NS = 16
_LANES = 16
_NW = _NC * _NS
_ROWS_PER_TILE = _N // _NW
_TILE_ELEMS = _ROWS_PER_TILE * _N
_GROUPS_PER_ROW = _N // _LANES


_EI_COLS = _N * _N


def _edge_index_body(out_ref):
    c = pl.program_id(0)
    col = lax.broadcasted_iota(jnp.int32, (2, _EI_COLS), 1) + c * _EI_COLS
    row = lax.broadcasted_iota(jnp.int32, (2, _EI_COLS), 0)
    v0 = col >> 9
    v1 = ((col >> 18) << 9) + (col & (_N - 1))
    out_ref[...] = jnp.where(row == 0, v0, v1)


_edge_index_tc = pl.pallas_call(
    _edge_index_body,
    grid=((_B * _N * _N) // _EI_COLS,),
    out_specs=pl.BlockSpec((2, _EI_COLS), lambda c: (0, c)),
    out_shape=jax.ShapeDtypeStruct((2, _B * _N * _N), jnp.int32),
)


def _edge_attr_body(adj_hbm, adjt_hbm, out_hbm, va, vb, sbuf, sem_in, sem_out):
    wid = lax.axis_index("s") * _NC + lax.axis_index("c")
    r0 = wid * _ROWS_PER_TILE
    cp_a = pltpu.make_async_copy(
        adj_hbm.at[pl.ds(r0, _ROWS_PER_TILE), :], va, sem_in)
    cp_b = pltpu.make_async_copy(
        adjt_hbm.at[pl.ds(r0, _ROWS_PER_TILE), :], vb, sem_in)
    cp_a.start()
    cp_b.start()
    cp_a.wait()
    cp_b.wait()

    def body(g, carry):
        i = g // _GROUPS_PER_ROW
        k = (g % _GROUPS_PER_ROW) * _LANES
        a = va[i, pl.ds(k, _LANES)]
        b = vb[i, pl.ds(k, _LANES)]
        sa = 1.0 / (1.0 + jnp.exp(-a))
        sb = 1.0 / (1.0 + jnp.exp(-b))
        sbuf[pl.ds(g * _LANES, _LANES)] = (sa + sb) * 0.5
        return carry

    lax.fori_loop(0, _ROWS_PER_TILE * _GROUPS_PER_ROW, body, 0)

    copies = []
    for b in range(_B):
        cp = pltpu.make_async_copy(
            sbuf,
            out_hbm.at[pl.ds(b * _N * _N + wid * _TILE_ELEMS, _TILE_ELEMS)],
            sem_out)
        cp.start()
        copies.append(cp)
    for cp in copies:
        cp.wait()


@functools.lru_cache(maxsize=1)
def _edge_attr_sc():
    return functools.partial(
        pl.kernel,
        out_type=jax.ShapeDtypeStruct((_B * _N * _N,), jnp.float32),
        mesh=plsc.VectorSubcoreMesh(core_axis_name="c", subcore_axis_name="s"),
        scratch_types=[
            pltpu.VMEM((_ROWS_PER_TILE, _N), jnp.float32),
            pltpu.VMEM((_ROWS_PER_TILE, _N), jnp.float32),
            pltpu.VMEM((_TILE_ELEMS,), jnp.float32),
            pltpu.SemaphoreType.DMA,
            pltpu.SemaphoreType.DMA,
        ],
    )(_edge_attr_body)


def kernel(x, adj):
    Bv, Nv, Dv = x.shape
    edge_index = _edge_index_tc()
    edge_attr = _edge_attr_sc()(adj, adj.T)
    x_batched = x.reshape(Bv * Nv, Dv)
    return x_batched, edge_index, edge_attr


# speedup vs baseline: 16.6629x; 1.5134x over previous
import functools

import jax
import jax.numpy as jnp
from jax import lax
from jax.experimental import pallas as pl
from jax.experimental.pallas import tpu as pltpu
from jax.experimental.pallas import tpu_sc as plsc

_B, _N, _D = 16, 512, 256
_NC = 2
_NS = 16
_LANES = 16
_NW = _NC * _NS
_ROWS_PER_TILE = _N // _NW
_TILE_ELEMS = _ROWS_PER_TILE * _N
_GROUPS_PER_ROW = _N // _LANES


_EI_COLS = _N * _N


def _edge_index_body(x_ref, ei_ref, xb_ref, pat_ref):
    c = pl.program_id(0)

    @pl.when(c == 0)
    def _():
        col = lax.broadcasted_iota(jnp.int32, (2, _EI_COLS), 1)
        row = lax.broadcasted_iota(jnp.int32, (2, _EI_COLS), 0)
        v0 = col >> 9
        v1 = col & (_N - 1)
        pat_ref[...] = jnp.where(row == 0, v0, v1)

    ei_ref[...] = pat_ref[...] + c * _N
    xb_ref[...] = x_ref[0]


_edge_index_tc = pl.pallas_call(
    _edge_index_body,
    grid=(_B,),
    in_specs=[pl.BlockSpec((1, _N, _D), lambda c: (c, 0, 0))],
    out_specs=[
        pl.BlockSpec((2, _EI_COLS), lambda c: (0, c)),
        pl.BlockSpec((_N, _D), lambda c: (c, 0)),
    ],
    out_shape=[
        jax.ShapeDtypeStruct((2, _B * _N * _N), jnp.int32),
        jax.ShapeDtypeStruct((_B * _N, _D), jnp.float32),
    ],
    scratch_shapes=[pltpu.VMEM((2, _EI_COLS), jnp.int32)],
)


def _edge_attr_body(adj_hbm, adjt_hbm, out_hbm, va, vb, sbuf, sem_in, sem_out):
    wid = lax.axis_index("s") * _NC + lax.axis_index("c")
    r0 = wid * _ROWS_PER_TILE
    cp_a = pltpu.make_async_copy(
        adj_hbm.at[pl.ds(r0, _ROWS_PER_TILE), :], va, sem_in)
    cp_b = pltpu.make_async_copy(
        adjt_hbm.at[pl.ds(r0, _ROWS_PER_TILE), :], vb, sem_in)
    cp_a.start()
    cp_b.start()
    cp_a.wait()
    cp_b.wait()

    def body(g, carry):
        i = g // _GROUPS_PER_ROW
        k = (g % _GROUPS_PER_ROW) * _LANES
        a = va[i, pl.ds(k, _LANES)]
        b = vb[i, pl.ds(k, _LANES)]
        sa = 1.0 / (1.0 + jnp.exp(-a))
        sb = 1.0 / (1.0 + jnp.exp(-b))
        sbuf[pl.ds(g * _LANES, _LANES)] = (sa + sb) * 0.5
        return carry

    lax.fori_loop(0, _ROWS_PER_TILE * _GROUPS_PER_ROW, body, 0)

    copies = []
    for b in range(_B):
        cp = pltpu.make_async_copy(
            sbuf,
            out_hbm.at[pl.ds(b * _N * _N + wid * _TILE_ELEMS, _TILE_ELEMS)],
            sem_out)
        cp.start()
        copies.append(cp)
    for cp in copies:
        cp.wait()


@functools.lru_cache(maxsize=1)
def _edge_attr_sc():
    return functools.partial(
        pl.kernel,
        out_type=jax.ShapeDtypeStruct((_B * _N * _N,), jnp.float32),
        mesh=plsc.VectorSubcoreMesh(core_axis_name="c", subcore_axis_name="s"),
        scratch_types=[
            pltpu.VMEM((_ROWS_PER_TILE, _N), jnp.float32),
            pltpu.VMEM((_ROWS_PER_TILE, _N), jnp.float32),
            pltpu.VMEM((_TILE_ELEMS,), jnp.float32),
            pltpu.SemaphoreType.DMA,
            pltpu.SemaphoreType.DMA,
        ],
    )(_edge_attr_body)


def kernel(x, adj):
    edge_index, x_batched = _edge_index_tc(x)
    edge_attr = _edge_attr_sc()(adj, adj.T)
    return x_batched, edge_index, edge_attr
